# Initial kernel scaffold; baseline (speedup 1.0000x reference)
import jax
import jax.numpy as jnp
from jax.experimental import pallas as pl


def kernel(pred_boxes, pred_scores, selected_indexes):
    raise NotImplementedError("write your pallas kernel here")


# speedup vs baseline: 2.5600x; 2.5600x over previous
import functools

import jax
import jax.numpy as jnp
from jax import lax
from jax.experimental import pallas as pl
from jax.experimental.pallas import tpu as pltpu
from jax.experimental.pallas import tpu_sc as plsc

B = 8
NPRE = 20000
MAXP = 1000
NCLS = 80
S = 8000
NS = 16
L = 16
SP = 8192
CH = SP // NS
NV = CH // L
ROWP = 1024
DUMP = 1008
NSEG = 4
SEG = CH // NSEG


def _prefix(v, iota):
    for st in (1, 2, 4, 8):
        sh = v[(iota - st) & (L - 1)]
        v = v + jnp.where(iota >= st, sh, 0)
    return v


def _sc_body(boxes_hbm, bat_hbm, lab_hbm, box_hbm,
             boxes_out, classes_out, num_out, sidx_out, didx_out,
             bat_v, lab_v, box_v, lpos_v, box4_v,
             b4idx_v, sidx_v, didx_v, d4idx_v, cnt16_v, negf_v, negi_v,
             cntall_v, shared_cnt, sbox, scls, gsem, ssem):
    w = lax.axis_index("s")
    base = w * CH
    iota = jax.lax.iota(jnp.int32, L)
    last = jnp.full((L,), L - 1, jnp.int32)
    wspl = jnp.broadcast_to(w, (L,))
    zero = jnp.zeros((L,), jnp.int32)

    pltpu.sync_copy(bat_hbm.at[pl.ds(base, CH)], bat_v)
    pltpu.sync_copy(lab_hbm.at[pl.ds(base, CH)], lab_v)
    pltpu.sync_copy(box_hbm.at[pl.ds(base, CH)], box_v)

    cnt = [zero] * B
    for i in range(NV):
        sl = pl.ds(i * L, L)
        bv = bat_v[sl]
        xv = box_v[sl]
        lv = lab_v[sl]
        bc = jnp.minimum(bv, B - 1)
        gb = bc * NPRE + xv
        gs = (bc * NCLS + lv) * NPRE + xv
        j, o = divmod(i * L, SEG)
        gb4 = bc * 4 * NPRE + xv
        for c in range(4):
            b4idx_v[c * NSEG + j, pl.ds(o, L)] = gb4 + c * NPRE
        sidx_v[j, pl.ds(o, L)] = gs
        lpos = zero
        for b in range(B):
            m = bv == b
            incl = _prefix(jnp.where(m, 1, 0), iota)
            lpos = jnp.where(m, cnt[b] + incl - 1, lpos)
            cnt[b] = cnt[b] + incl[last]
        lpos_v[sl] = lpos

    cv = zero
    for b in range(B):
        cv = jnp.where(iota == b, cnt[b], cv)
    cnt16_v[...] = cv
    pltpu.sync_copy(cnt16_v.at[pl.ds(0, B)], shared_cnt.at[pl.ds(w * B, B)])

    gdescs = []
    for j in range(NSEG):
        for c in range(4):
            gdescs.append(pltpu.async_copy(
                boxes_hbm.at[b4idx_v.at[c * NSEG + j]],
                box4_v.at[pl.ds(c * CH + j * SEG, SEG)], gsem))

    negv = jnp.full((L,), -1.0, jnp.float32)
    negvi = jnp.full((L,), -1, jnp.int32)
    for k in range(NV):
        negf_v[pl.ds(k * L, L)] = negv
        negi_v[pl.ds(k * L, L)] = negvi
    for q in range(4):
        pltpu.sync_copy(negf_v, sbox.at[pl.ds(base * 4 + q * CH, CH)])
    pltpu.sync_copy(negi_v, scls.at[pl.ds(base, CH)])

    plsc.subcore_barrier()

    pltpu.sync_copy(shared_cnt, cntall_v)
    mybase = [None] * B
    tot = [None] * B
    for b in range(B):
        idxc = jnp.where((iota & 1) == 1, b + B, b)
        vb = zero
        for k in range(B):
            s = cntall_v[pl.ds(k * L, L)]
            vb = jnp.where((iota >> 1) == k, s[idxc], vb)
        incl = _prefix(vb, iota)
        excl = incl - vb
        mybase[b] = excl[wspl]
        tot[b] = incl[last]

    @pl.when(w == 0)
    def _():
        npv = zero
        for b in range(B):
            npv = jnp.where(iota == b, jnp.minimum(tot[b], MAXP), npv)
        cnt16_v[...] = npv
        pltpu.sync_copy(cnt16_v.at[pl.ds(0, B)], num_out)

    for i in range(NV):
        sl = pl.ds(i * L, L)
        bv = bat_v[sl]
        pos = lpos_v[sl]
        for b in range(B):
            pos = jnp.where(bv == b, pos + mybase[b], pos)
        d = (jnp.minimum(bv, B - 1) * ROWP
             + jnp.minimum(jnp.where(bv < B, pos, DUMP), DUMP))
        j, o = divmod(i * L, SEG)
        didx_v[j, pl.ds(o, L)] = d
        d4 = d * 4
        for c in range(4):
            d4idx_v[c * NSEG + j, pl.ds(o, L)] = d4 + c

    pltpu.sync_copy(sidx_v, sidx_out.at[pl.ds(w * NSEG, NSEG)])
    pltpu.sync_copy(didx_v, didx_out.at[pl.ds(w * NSEG, NSEG)])

    for dsc in gdescs:
        dsc.wait()

    sdescs = []
    for j in range(NSEG):
        seg = pl.ds(j * SEG, SEG)
        di = didx_v.at[j]
        for c in range(4):
            sdescs.append(pltpu.async_copy(
                box4_v.at[pl.ds(c * CH + j * SEG, SEG)],
                sbox.at[d4idx_v.at[c * NSEG + j]], ssem))
        sdescs.append(pltpu.async_copy(lab_v.at[seg], scls.at[di], ssem))
    for dsc in sdescs:
        dsc.wait()

    plsc.subcore_barrier()

    pltpu.sync_copy(sbox.at[pl.ds(base * 4, CH * 4)],
                    boxes_out.at[pl.ds(base * 4, CH * 4)])
    pltpu.sync_copy(scls.at[pl.ds(base, CH)], classes_out.at[pl.ds(base, CH)])


_sc_call = functools.partial(
    pl.kernel,
    out_type=[
        jax.ShapeDtypeStruct((B * ROWP * 4,), jnp.float32),
        jax.ShapeDtypeStruct((B * ROWP,), jnp.int32),
        jax.ShapeDtypeStruct((B,), jnp.int32),
        jax.ShapeDtypeStruct((NS * NSEG, SEG), jnp.int32),
        jax.ShapeDtypeStruct((NS * NSEG, SEG), jnp.int32),
    ],
    mesh=plsc.VectorSubcoreMesh(core_axis_name="c", subcore_axis_name="s",
                                num_cores=1),
    compiler_params=pltpu.CompilerParams(use_tc_tiling_on_sc=False),
    scratch_types=[
        pltpu.VMEM((CH,), jnp.int32),
        pltpu.VMEM((CH,), jnp.int32),
        pltpu.VMEM((CH,), jnp.int32),
        pltpu.VMEM((CH,), jnp.int32),
        pltpu.VMEM((CH * 4,), jnp.float32),
        pltpu.VMEM((4 * NSEG, SEG), jnp.int32),
        pltpu.VMEM((NSEG, SEG), jnp.int32),
        pltpu.VMEM((NSEG, SEG), jnp.int32),
        pltpu.VMEM((4 * NSEG, SEG), jnp.int32),
        pltpu.VMEM((L,), jnp.int32),
        pltpu.VMEM((CH,), jnp.float32),
        pltpu.VMEM((CH,), jnp.int32),
        pltpu.VMEM((NS * B,), jnp.int32),
        pltpu.VMEM_SHARED((NS * B,), jnp.int32),
        pltpu.VMEM_SHARED((SP * 4,), jnp.float32),
        pltpu.VMEM_SHARED((SP,), jnp.int32),
        pltpu.SemaphoreType.DMA,
        pltpu.SemaphoreType.DMA,
    ],
)(_sc_body)


def _sc_body_scores(scores_hbm, sidx_hbm, didx_hbm, scores_out,
                    sidx_v, didx_v, scog_v, negf_v, ssco, gsem, ssem):
    w = lax.axis_index("s")
    base = w * CH
    iota = jax.lax.iota(jnp.int32, L)

    pltpu.sync_copy(sidx_hbm.at[pl.ds(w * NSEG, NSEG)], sidx_v)
    pltpu.sync_copy(didx_hbm.at[pl.ds(w * NSEG, NSEG)], didx_v)

    gdescs = []
    for j in range(NSEG):
        gdescs.append(pltpu.async_copy(
            scores_hbm.at[sidx_v.at[j]], scog_v.at[pl.ds(j * SEG, SEG)], gsem))

    negv = jnp.full((L,), -1.0, jnp.float32)
    for k in range(NV):
        negf_v[pl.ds(k * L, L)] = negv
    pltpu.sync_copy(negf_v, ssco.at[pl.ds(base, CH)])

    plsc.subcore_barrier()

    for dsc in gdescs:
        dsc.wait()
    sdescs = []
    for j in range(NSEG):
        sdescs.append(pltpu.async_copy(
            scog_v.at[pl.ds(j * SEG, SEG)], ssco.at[didx_v.at[j]], ssem))
    for dsc in sdescs:
        dsc.wait()

    plsc.subcore_barrier()
    pltpu.sync_copy(ssco.at[pl.ds(base, CH)], scores_out.at[pl.ds(base, CH)])


_sc_scores_call = functools.partial(
    pl.kernel,
    out_type=[jax.ShapeDtypeStruct((B * ROWP,), jnp.float32)],
    mesh=plsc.VectorSubcoreMesh(core_axis_name="c", subcore_axis_name="s",
                                num_cores=1),
    compiler_params=pltpu.CompilerParams(use_tc_tiling_on_sc=False),
    scratch_types=[
        pltpu.VMEM((NSEG, SEG), jnp.int32),
        pltpu.VMEM((NSEG, SEG), jnp.int32),
        pltpu.VMEM((CH,), jnp.float32),
        pltpu.VMEM((CH,), jnp.float32),
        pltpu.VMEM_SHARED((SP,), jnp.float32),
        pltpu.SemaphoreType.DMA,
        pltpu.SemaphoreType.DMA,
    ],
)(_sc_body_scores)


def kernel(pred_boxes, pred_scores, selected_indexes):
    boxes_flat = jnp.swapaxes(pred_boxes, 1, 2).reshape(B * NPRE * 4)
    scores_flat = jnp.swapaxes(pred_scores, 1, 2).reshape(B * NPRE * NCLS)
    bat = selected_indexes[:, 0]
    lab = selected_indexes[:, 1]
    box = selected_indexes[:, 2]
    pad = SP - S
    bat_p = jnp.concatenate([bat, jnp.full((pad,), B, jnp.int32)])
    lab_p = jnp.concatenate([lab, jnp.zeros((pad,), jnp.int32)])
    box_p = jnp.concatenate([box, jnp.zeros((pad,), jnp.int32)])
    boxes_o, classes_o, nump, sidx_o, didx_o = _sc_call(
        boxes_flat, bat_p, lab_p, box_p)
    (scores_o,) = _sc_scores_call(scores_flat, sidx_o, didx_o)
    return (
        nump.reshape(B, 1),
        boxes_o.reshape(B, ROWP, 4)[:, :MAXP],
        scores_o.reshape(B, ROWP)[:, :MAXP],
        classes_o.reshape(B, ROWP)[:, :MAXP],
    )
